# baseline (speedup 1.0000x reference)
import functools

import jax
import jax.numpy as jnp
from jax import lax
from jax.experimental import pallas as pl
from jax.experimental.pallas import tpu as pltpu
from jax.experimental.pallas import tpu_sc as plsc

_CODEBOOK = 1024
_DIM = 64
_ROWS = 16384
_R = 512
_G = _ROWS // _R
_HALVES = 1024 // _R
_BETA = 0.25

_NC = 2
_NS = 16
_NW = _NC * _NS
_BPW = _ROWS // _NW


def _argmin_body(x_ref, emb_ref, se_ref, idx_ref, loss_ref):
    x = x_ref[...].reshape(_R, _DIM)
    emb = emb_ref[...]
    sx = jnp.sum(x * x, axis=1, keepdims=True)
    t = lax.dot_general(x, emb, (((1,), (1,)), ((), ())),
                        preferred_element_type=jnp.float32)
    d = (sx - 2.0 * t) + se_ref[...]
    m = jnp.min(d, axis=1, keepdims=True)
    lanes = lax.broadcasted_iota(jnp.int32, d.shape, 1)
    idx_ref[...] = jnp.min(jnp.where(d == m, lanes, _CODEBOOK), axis=1,
                           keepdims=True)
    loss_ref[...] = jnp.sum(m).reshape(1, 1, 1)


def _tc_argmin(z_e3, se, e):
    return pl.pallas_call(
        _argmin_body,
        grid=(_G,),
        in_specs=[
            pl.BlockSpec((1, _R, _DIM),
                         lambda i: (i // _HALVES, i % _HALVES, 0)),
            pl.BlockSpec((_CODEBOOK, _DIM), lambda i: (0, 0)),
            pl.BlockSpec((1, _CODEBOOK), lambda i: (0, 0)),
        ],
        out_specs=[
            pl.BlockSpec((_R, 1), lambda i: (i, 0)),
            pl.BlockSpec((1, 1, 1), lambda i: (i, 0, 0)),
        ],
        out_shape=[
            jax.ShapeDtypeStruct((_ROWS, 1), jnp.int32),
            jax.ShapeDtypeStruct((_G, 1, 1), jnp.float32),
        ],
        compiler_params=pltpu.CompilerParams(
            dimension_semantics=("parallel",)),
    )(z_e3, e, se)


@functools.partial(
    pl.kernel,
    out_type=jax.ShapeDtypeStruct((_ROWS, _DIM), jnp.float32),
    mesh=plsc.VectorSubcoreMesh(core_axis_name="c", subcore_axis_name="s",
                                num_cores=_NC, num_subcores=_NS),
    scratch_types=[
        pltpu.VMEM((_BPW,), jnp.int32),
        pltpu.VMEM((_BPW, _DIM), jnp.float32),
        pltpu.SemaphoreType.DMA,
    ],
    compiler_params=pltpu.CompilerParams(use_tc_tiling_on_sc=False),
)
def _sc_gather(emb_hbm, idx_hbm, out_hbm, idx_v, rows_v, sem):
    wid = lax.axis_index("s") * _NC + lax.axis_index("c")
    base = wid * _BPW
    pltpu.sync_copy(idx_hbm.at[pl.ds(base, _BPW)], idx_v)
    pltpu.async_copy(emb_hbm.at[idx_v], rows_v, sem).wait()
    pltpu.sync_copy(rows_v, out_hbm.at[pl.ds(base, _BPW)])


def kernel(z_e, emb):
    bsz, num_slots, code_dim = z_e.shape
    z_e3 = z_e.astype(jnp.float32)
    e = emb.astype(jnp.float32)
    se = jnp.sum(e ** 2, axis=1, keepdims=True).T

    idx, loss_parts = _tc_argmin(z_e3, se, e)
    idx_flat = idx.reshape(_ROWS)
    zq = _sc_gather(e, idx_flat)
    loss = jnp.sum(loss_parts) * ((1.0 + _BETA) / float(_ROWS * _DIM))

    return (zq.reshape(bsz, num_slots, code_dim),
            idx.reshape(bsz, num_slots),
            loss)


# speedup vs baseline: 1.8310x; 1.8310x over previous
import jax
import jax.numpy as jnp
from jax import lax
from jax.experimental import pallas as pl
from jax.experimental.pallas import tpu as pltpu

_CODEBOOK = 1024
_DIM = 64
_ROWS = 16384
_R = 512
_G = _ROWS // _R
_HALVES = 1024 // _R
_BETA = 0.25


def _vq_body(xt_ref, emb_ref, embt_ref, idx_ref, out_ref, loss_ref):
    xt = xt_ref[...].reshape(_DIM, _R)
    sx = jnp.sum(xt * xt, axis=0, keepdims=True)
    tt = lax.dot_general(emb_ref[...], xt, (((1,), (0,)), ((), ())),
                         preferred_element_type=jnp.float32)
    se = jnp.sum(emb_ref[...] ** 2, axis=1, keepdims=True)
    d = (sx - 2.0 * tt) + se
    m = jnp.min(d, axis=0, keepdims=True)
    codes = lax.broadcasted_iota(jnp.int32, d.shape, 0)
    idx = jnp.min(jnp.where(d == m, codes, _CODEBOOK), axis=0,
                  keepdims=True)
    idx_ref[...] = idx.reshape(1, 1, _R)
    onehot = (codes == idx).astype(jnp.float32)
    zqt = lax.dot_general(embt_ref[...], onehot, (((1,), (0,)), ((), ())),
                          preferred_element_type=jnp.float32)
    out_ref[...] = (xt + (zqt - xt)).reshape(1, _DIM, _R)
    loss_ref[...] = jnp.sum(m).reshape(1, 1, 1)


def kernel(z_e, emb):
    bsz, num_slots, code_dim = z_e.shape
    xt = lax.transpose(z_e.astype(jnp.float32), (0, 2, 1))
    e = emb.astype(jnp.float32)
    et = e.T

    idx3, out_t, loss_parts = pl.pallas_call(
        _vq_body,
        grid=(_G,),
        in_specs=[
            pl.BlockSpec((1, _DIM, _R),
                         lambda i: (i // _HALVES, 0, i % _HALVES)),
            pl.BlockSpec((_CODEBOOK, _DIM), lambda i: (0, 0)),
            pl.BlockSpec((_DIM, _CODEBOOK), lambda i: (0, 0)),
        ],
        out_specs=[
            pl.BlockSpec((1, 1, _R), lambda i: (i // _HALVES, 0, i % _HALVES)),
            pl.BlockSpec((1, _DIM, _R),
                         lambda i: (i // _HALVES, 0, i % _HALVES)),
            pl.BlockSpec((1, 1, 1), lambda i: (i, 0, 0)),
        ],
        out_shape=[
            jax.ShapeDtypeStruct((bsz, 1, num_slots), jnp.int32),
            jax.ShapeDtypeStruct((bsz, _DIM, num_slots), jnp.float32),
            jax.ShapeDtypeStruct((_G, 1, 1), jnp.float32),
        ],
        compiler_params=pltpu.CompilerParams(
            dimension_semantics=("parallel",)),
    )(xt, e, et)

    loss = jnp.sum(loss_parts) * ((1.0 + _BETA) / float(_ROWS * _DIM))
    return (lax.transpose(out_t, (0, 2, 1)),
            idx3.reshape(bsz, num_slots),
            loss)
